# baseline (speedup 1.0000x reference)
import functools

import jax
import jax.numpy as jnp
from jax import lax
from jax.experimental import pallas as pl
from jax.experimental.pallas import tpu as pltpu
from jax.experimental.pallas import tpu_sc as plsc

_N_NODES = 10000
_D = 128
_N_GRAPHS = 512
_N_CONV = 3

_BLK = 1000
_NB = _N_NODES // _BLK

_NC = 2
_NS = 16
_CH = 96
_C = 107
_EPAD = _NC * _NS * _C * _CH
_NPAD = 10240
_RPT = _NPAD // _NS
_FB = 80
_NF = _RPT // _FB
_DUMMY = _N_NODES


def _softplus(z):
    return jnp.maximum(z, 0.0) + jnp.log1p(jnp.exp(-jnp.abs(z)))


def _sigmoid(z):
    return 1.0 / (1.0 + jnp.exp(-z))


def _dot_t(a, b):
    return lax.dot_general(a, b, (((1,), (1,)), ((), ())),
                           preferred_element_type=jnp.float32)


def _embed_msg_body(nodes_ref, embw_ref, w1_ref, w2_ref, x_ref, m_ref):
    xb = _dot_t(nodes_ref[...], embw_ref[...])
    z1 = _dot_t(xb, w1_ref[...])
    z2 = _dot_t(xb, w2_ref[...])
    x_ref[...] = xb
    m_ref[...] = _sigmoid(z1) * _softplus(z2)


def _update_msg_body(x_ref, agg0_ref, agg1_ref, w1_ref, w2_ref, xn_ref, m_ref):
    xn = x_ref[...] + _softplus(agg0_ref[0] + agg1_ref[0])
    z1 = _dot_t(xn, w1_ref[...])
    z2 = _dot_t(xn, w2_ref[...])
    xn_ref[...] = xn
    m_ref[...] = _sigmoid(z1) * _softplus(z2)


def _final_body(x_ref, agg0_ref, agg1_ref, gidx_ref, invc_ref, fcw_ref,
                fcb_ref, regw_ref, regb_ref, out_ref, acc_ref):
    i = pl.program_id(0)

    @pl.when(i == 0)
    def _():
        acc_ref[...] = jnp.zeros_like(acc_ref)

    xb = x_ref[...] + _softplus(agg0_ref[0] + agg1_ref[0])
    g = gidx_ref[0]
    iota = lax.broadcasted_iota(jnp.int32, (_N_GRAPHS, _BLK), 0)
    onehot = (iota == g).astype(jnp.float32)
    acc_ref[...] += lax.dot_general(onehot, xb, (((1,), (0,)), ((), ())),
                                    preferred_element_type=jnp.float32)

    @pl.when(i == _NB - 1)
    def _():
        pooled = acc_ref[...] * invc_ref[...]
        y = _softplus(_dot_t(pooled, fcw_ref[0]) + fcb_ref[0])
        y = _softplus(_dot_t(y, fcw_ref[1]) + fcb_ref[1])
        out_ref[...] = (jnp.sum(y * regw_ref[...], axis=1, keepdims=True)
                        + regb_ref[...])


def _tc_embed_msg(nodes, emb_w, w1, w2):
    full = lambda i: (0, 0)
    blk = lambda i: (i, 0)
    return pl.pallas_call(
        _embed_msg_body,
        grid=(_NB,),
        in_specs=[
            pl.BlockSpec((_BLK, _D), blk),
            pl.BlockSpec((_D, _D), full),
            pl.BlockSpec((_D, _D), full),
            pl.BlockSpec((_D, _D), full),
        ],
        out_specs=[
            pl.BlockSpec((_BLK, _D), blk),
            pl.BlockSpec((_BLK, _D), blk),
        ],
        out_shape=[
            jax.ShapeDtypeStruct((_N_NODES, _D), jnp.float32),
            jax.ShapeDtypeStruct((_N_NODES, _D), jnp.float32),
        ],
    )(nodes, emb_w, w1, w2)


def _tc_update_msg(x, partials, w1, w2):
    full = lambda i: (0, 0)
    blk = lambda i: (i, 0)
    return pl.pallas_call(
        _update_msg_body,
        grid=(_NB,),
        in_specs=[
            pl.BlockSpec((_BLK, _D), blk),
            pl.BlockSpec((1, _BLK, _D), lambda i: (0, i, 0)),
            pl.BlockSpec((1, _BLK, _D), lambda i: (1, i, 0)),
            pl.BlockSpec((_D, _D), full),
            pl.BlockSpec((_D, _D), full),
        ],
        out_specs=[
            pl.BlockSpec((_BLK, _D), blk),
            pl.BlockSpec((_BLK, _D), blk),
        ],
        out_shape=[
            jax.ShapeDtypeStruct((_N_NODES, _D), jnp.float32),
            jax.ShapeDtypeStruct((_N_NODES, _D), jnp.float32),
        ],
    )(x, partials, partials, w1, w2)


def _tc_final(x, partials, gidx3d, inv_counts, fc_w, fc_b3d, reg_w, reg_b2d):
    blk = lambda i: (i, 0)
    return pl.pallas_call(
        _final_body,
        grid=(_NB,),
        in_specs=[
            pl.BlockSpec((_BLK, _D), blk),
            pl.BlockSpec((1, _BLK, _D), lambda i: (0, i, 0)),
            pl.BlockSpec((1, _BLK, _D), lambda i: (1, i, 0)),
            pl.BlockSpec((1, 1, _BLK), lambda i: (i, 0, 0)),
            pl.BlockSpec((_N_GRAPHS, 1), lambda i: (0, 0)),
            pl.BlockSpec((2, _D, _D), lambda i: (0, 0, 0)),
            pl.BlockSpec((2, 1, _D), lambda i: (0, 0, 0)),
            pl.BlockSpec((1, _D), lambda i: (0, 0)),
            pl.BlockSpec((1, 1), lambda i: (0, 0)),
        ],
        out_specs=pl.BlockSpec((_N_GRAPHS, 1), lambda i: (0, 0)),
        out_shape=jax.ShapeDtypeStruct((_N_GRAPHS, 1), jnp.float32),
        scratch_shapes=[pltpu.VMEM((_N_GRAPHS, _D), jnp.float32)],
    )(x, partials, partials, gidx3d, inv_counts, fc_w, fc_b3d, reg_w, reg_b2d)


def _sc_agg(m, pk_idx):
    mesh = plsc.VectorSubcoreMesh(core_axis_name="c", subcore_axis_name="s")

    @functools.partial(
        pl.kernel,
        out_type=jax.ShapeDtypeStruct((_NC, _NPAD, _D), jnp.float32),
        mesh=mesh,
        scratch_types=[
            pltpu.VMEM((_C, _CH), jnp.int32),
            pltpu.VMEM((2, _CH), jnp.int32),
            pltpu.VMEM((2, _CH), jnp.int32),
            pltpu.VMEM((_CH, _D), jnp.float32),
            pltpu.VMEM((_CH, _D), jnp.float32),
            pltpu.VMEM_SHARED((_NPAD, _D), jnp.float32),
            pltpu.SemaphoreType.DMA,
            pltpu.SemaphoreType.DMA,
        ],
    )
    def k(m_hbm, pk_hbm, out_hbm, pk_v, s_sc, t_sc, rows0_v, rows1_v,
          agg_sh, sem0, sem1):
        cid = lax.axis_index("c")
        sid = lax.axis_index("s")
        base = sid * _RPT

        def unpack(c, slot):
            for kk in range(_CH // 16):
                p = pk_v[c, pl.ds(kk * 16, 16)]
                s_sc[slot, pl.ds(kk * 16, 16)] = jnp.bitwise_and(p, 0xFFFF)
                t_sc[slot, pl.ds(kk * 16, 16)] = lax.shift_right_logical(p, 16)

        def zrow(i, carry):
            for kk in range(8):
                rows0_v[i, pl.ds(kk * 16, 16)] = jnp.zeros((16,), jnp.float32)
            return carry
        lax.fori_loop(0, _FB, zrow, 0)
        for f in range(_NF):
            pltpu.sync_copy(rows0_v.at[pl.ds(0, _FB)],
                            agg_sh.at[pl.ds(base + f * _FB, _FB)])

        pltpu.sync_copy(pk_hbm.at[cid, sid], pk_v)

        plsc.subcore_barrier()

        unpack(0, 0)
        pltpu.async_copy(m_hbm.at[s_sc.at[0]], rows0_v, sem0)

        def step(j, carry):
            for par in range(2):
                rv = rows0_v if par == 0 else rows1_v
                sm = sem0 if par == 0 else sem1
                nrv = rows1_v if par == 0 else rows0_v
                nsm = sem1 if par == 0 else sem0
                jj = 2 * j + par
                unpack(jj + 1, 1 - par)
                pltpu.async_copy(m_hbm.at[s_sc.at[1 - par]], nrv, nsm)
                pltpu.make_async_copy(m_hbm.at[s_sc.at[par]], rv, sm).wait()
                pltpu.sync_copy(rv, agg_sh.at[t_sc.at[par]], add=True)
            return carry
        lax.fori_loop(0, (_C - 1) // 2, step, 0)
        pltpu.make_async_copy(m_hbm.at[s_sc.at[0]], rows0_v, sem0).wait()
        pltpu.sync_copy(rows0_v, agg_sh.at[t_sc.at[0]], add=True)

        plsc.subcore_barrier()

        for f in range(_NF):
            pltpu.sync_copy(agg_sh.at[pl.ds(base + f * _FB, _FB)],
                            rows0_v.at[pl.ds(0, _FB)])
            pltpu.sync_copy(rows0_v.at[pl.ds(0, _FB)],
                            out_hbm.at[cid].at[pl.ds(base + f * _FB, _FB)])

    return k(m, pk_idx)


def kernel(nodes, node_counts, edge_sources, edge_targets, graph_indices,
           emb_W, conv_W1, conv_W2, fc_W, fc_b, reg_W, reg_b):
    pad = _EPAD - edge_sources.shape[0]
    pk = edge_targets * 65536 + edge_sources
    pk_p = jnp.concatenate(
        [pk, jnp.full((pad,), _DUMMY * 65536, jnp.int32)]
    ).reshape(_NC, _NS, _C, _CH)

    x, m = _tc_embed_msg(nodes, emb_W, conv_W1[0], conv_W2[0])
    for i in range(_N_CONV):
        partials = _sc_agg(m, pk_p)
        if i + 1 < _N_CONV:
            x, m = _tc_update_msg(x, partials, conv_W1[i + 1], conv_W2[i + 1])

    gidx3d = graph_indices.reshape(_NB, 1, _BLK)
    inv_counts = (1.0 / node_counts).reshape(_N_GRAPHS, 1)
    out2d = _tc_final(x, partials, gidx3d, inv_counts, fc_W,
                      fc_b.reshape(2, 1, _D), reg_W, reg_b.reshape(1, 1))
    return out2d[:, 0]


# speedup vs baseline: 1.0142x; 1.0142x over previous
import functools

import jax
import jax.numpy as jnp
from jax import lax
from jax.experimental import pallas as pl
from jax.experimental.pallas import tpu as pltpu
from jax.experimental.pallas import tpu_sc as plsc

_N_NODES = 10000
_D = 128
_N_GRAPHS = 512
_N_CONV = 3

_BLK = 1000
_NB = _N_NODES // _BLK

_NC = 2
_NS = 16
_CH = 128
_C = 80
_EPAD = _NC * _NS * _C * _CH
_NPAD = 10240
_RPT = _NPAD // _NS
_DUMMY = _N_NODES


def _softplus(z):
    return jnp.maximum(z, 0.0) + jnp.log1p(jnp.exp(-jnp.abs(z)))


def _sigmoid(z):
    return 1.0 / (1.0 + jnp.exp(-z))


def _dot_t(a, b):
    return lax.dot_general(a, b, (((1,), (1,)), ((), ())),
                           preferred_element_type=jnp.float32)


def _embed_msg_body(nodes_ref, embw_ref, w1_ref, w2_ref, x_ref, m_ref):
    xb = _dot_t(nodes_ref[...], embw_ref[...])
    z1 = _dot_t(xb, w1_ref[...])
    z2 = _dot_t(xb, w2_ref[...])
    x_ref[...] = xb
    m_ref[...] = _sigmoid(z1) * _softplus(z2)


def _update_msg_body(x_ref, agg0_ref, agg1_ref, w1_ref, w2_ref, xn_ref, m_ref):
    xn = x_ref[...] + _softplus(agg0_ref[0] + agg1_ref[0])
    z1 = _dot_t(xn, w1_ref[...])
    z2 = _dot_t(xn, w2_ref[...])
    xn_ref[...] = xn
    m_ref[...] = _sigmoid(z1) * _softplus(z2)


def _final_body(x_ref, agg0_ref, agg1_ref, gidx_ref, invc_ref, fcw_ref,
                fcb_ref, regw_ref, regb_ref, out_ref, acc_ref):
    i = pl.program_id(0)

    @pl.when(i == 0)
    def _():
        acc_ref[...] = jnp.zeros_like(acc_ref)

    xb = x_ref[...] + _softplus(agg0_ref[0] + agg1_ref[0])
    g = gidx_ref[0]
    iota = lax.broadcasted_iota(jnp.int32, (_N_GRAPHS, _BLK), 0)
    onehot = (iota == g).astype(jnp.float32)
    acc_ref[...] += lax.dot_general(onehot, xb, (((1,), (0,)), ((), ())),
                                    preferred_element_type=jnp.float32)

    @pl.when(i == _NB - 1)
    def _():
        pooled = acc_ref[...] * invc_ref[...]
        y = _softplus(_dot_t(pooled, fcw_ref[0]) + fcb_ref[0])
        y = _softplus(_dot_t(y, fcw_ref[1]) + fcb_ref[1])
        out_ref[...] = (jnp.sum(y * regw_ref[...], axis=1, keepdims=True)
                        + regb_ref[...])


def _tc_embed_msg(nodes, emb_w, w1, w2):
    full = lambda i: (0, 0)
    blk = lambda i: (i, 0)
    return pl.pallas_call(
        _embed_msg_body,
        grid=(_NB,),
        in_specs=[
            pl.BlockSpec((_BLK, _D), blk),
            pl.BlockSpec((_D, _D), full),
            pl.BlockSpec((_D, _D), full),
            pl.BlockSpec((_D, _D), full),
        ],
        out_specs=[
            pl.BlockSpec((_BLK, _D), blk),
            pl.BlockSpec((_BLK, _D), blk),
        ],
        out_shape=[
            jax.ShapeDtypeStruct((_N_NODES, _D), jnp.float32),
            jax.ShapeDtypeStruct((_N_NODES, _D), jnp.float32),
        ],
    )(nodes, emb_w, w1, w2)


def _tc_update_msg(x, partials, w1, w2):
    full = lambda i: (0, 0)
    blk = lambda i: (i, 0)
    return pl.pallas_call(
        _update_msg_body,
        grid=(_NB,),
        in_specs=[
            pl.BlockSpec((_BLK, _D), blk),
            pl.BlockSpec((1, _BLK, _D), lambda i: (0, i, 0)),
            pl.BlockSpec((1, _BLK, _D), lambda i: (1, i, 0)),
            pl.BlockSpec((_D, _D), full),
            pl.BlockSpec((_D, _D), full),
        ],
        out_specs=[
            pl.BlockSpec((_BLK, _D), blk),
            pl.BlockSpec((_BLK, _D), blk),
        ],
        out_shape=[
            jax.ShapeDtypeStruct((_N_NODES, _D), jnp.float32),
            jax.ShapeDtypeStruct((_N_NODES, _D), jnp.float32),
        ],
    )(x, partials, partials, w1, w2)


def _tc_final(x, partials, gidx3d, inv_counts, fc_w, fc_b3d, reg_w, reg_b2d):
    blk = lambda i: (i, 0)
    return pl.pallas_call(
        _final_body,
        grid=(_NB,),
        in_specs=[
            pl.BlockSpec((_BLK, _D), blk),
            pl.BlockSpec((1, _BLK, _D), lambda i: (0, i, 0)),
            pl.BlockSpec((1, _BLK, _D), lambda i: (1, i, 0)),
            pl.BlockSpec((1, 1, _BLK), lambda i: (i, 0, 0)),
            pl.BlockSpec((_N_GRAPHS, 1), lambda i: (0, 0)),
            pl.BlockSpec((2, _D, _D), lambda i: (0, 0, 0)),
            pl.BlockSpec((2, 1, _D), lambda i: (0, 0, 0)),
            pl.BlockSpec((1, _D), lambda i: (0, 0)),
            pl.BlockSpec((1, 1), lambda i: (0, 0)),
        ],
        out_specs=pl.BlockSpec((_N_GRAPHS, 1), lambda i: (0, 0)),
        out_shape=jax.ShapeDtypeStruct((_N_GRAPHS, 1), jnp.float32),
        scratch_shapes=[pltpu.VMEM((_N_GRAPHS, _D), jnp.float32)],
    )(x, partials, partials, gidx3d, inv_counts, fc_w, fc_b3d, reg_w, reg_b2d)


def _sc_agg(m, zeros_acc, st_idx):
    mesh = plsc.VectorSubcoreMesh(core_axis_name="c", subcore_axis_name="s")

    @functools.partial(
        pl.kernel,
        out_type=jax.ShapeDtypeStruct((_NC, _NPAD, _D), jnp.float32),
        mesh=mesh,
        scratch_types=[
            pltpu.VMEM((4, 2, _CH), jnp.int32),
            pltpu.VMEM((_CH, _D), jnp.float32),
            pltpu.VMEM((_CH, _D), jnp.float32),
            pltpu.VMEM_SHARED((_NPAD, _D), jnp.float32),
            pltpu.SemaphoreType.DMA,
            pltpu.SemaphoreType.DMA,
            pltpu.SemaphoreType.DMA,
            pltpu.SemaphoreType.DMA,
            pltpu.SemaphoreType.DMA,
            pltpu.SemaphoreType.DMA,
        ],
    )
    def k(m_hbm, z_hbm, st_hbm, out_hbm, st_v, rows0_v, rows1_v, agg_sh,
          gs0, gs1, is0, is1, is2, is3):
        cid = lax.axis_index("c")
        sid = lax.axis_index("s")
        base = sid * _RPT
        isems = (is0, is1, is2, is3)
        gsems = (gs0, gs1)
        rows = (rows0_v, rows1_v)

        pltpu.sync_copy(z_hbm.at[pl.ds(base, _RPT)],
                        agg_sh.at[pl.ds(base, _RPT)])

        def iload(c):
            pltpu.async_copy(st_hbm.at[cid, sid, c], st_v.at[c % 4],
                             isems[c % 4])

        def iwait(c):
            pltpu.make_async_copy(st_hbm.at[cid, sid, c], st_v.at[c % 4],
                                  isems[c % 4]).wait()

        def gissue(c):
            pltpu.async_copy(m_hbm.at[st_v.at[c % 4, 0]], rows[c % 2],
                             gsems[c % 2])

        def gwait(c):
            pltpu.make_async_copy(m_hbm.at[st_v.at[c % 4, 0]], rows[c % 2],
                                  gsems[c % 2]).wait()

        plsc.subcore_barrier()

        iload(0)
        iload(1)
        iwait(0)
        gissue(0)
        for c in range(_C):
            if c + 2 < _C:
                iload(c + 2)
            if c + 1 < _C:
                iwait(c + 1)
                gissue(c + 1)
            gwait(c)
            pltpu.sync_copy(rows[c % 2], agg_sh.at[st_v.at[c % 4, 1]],
                            add=True)

        plsc.subcore_barrier()

        pltpu.sync_copy(agg_sh.at[pl.ds(base, _RPT)],
                        out_hbm.at[cid].at[pl.ds(base, _RPT)])

    return k(m, zeros_acc, st_idx)


def kernel(nodes, node_counts, edge_sources, edge_targets, graph_indices,
           emb_W, conv_W1, conv_W2, fc_W, fc_b, reg_W, reg_b):
    pad = _EPAD - edge_sources.shape[0]
    src_p = jnp.concatenate(
        [edge_sources, jnp.zeros((pad,), jnp.int32)]).reshape(_NC, _NS, _C, _CH)
    tgt_p = jnp.concatenate(
        [edge_targets,
         jnp.full((pad,), _DUMMY, jnp.int32)]).reshape(_NC, _NS, _C, _CH)
    st_p = jnp.stack([src_p, tgt_p], axis=3)
    zeros_acc = jnp.zeros((_NPAD, _D), jnp.float32)

    x, m = _tc_embed_msg(nodes, emb_W, conv_W1[0], conv_W2[0])
    for i in range(_N_CONV):
        partials = _sc_agg(m, zeros_acc, st_p)
        if i + 1 < _N_CONV:
            x, m = _tc_update_msg(x, partials, conv_W1[i + 1], conv_W2[i + 1])

    gidx3d = graph_indices.reshape(_NB, 1, _BLK)
    inv_counts = (1.0 / node_counts).reshape(_N_GRAPHS, 1)
    out2d = _tc_final(x, partials, gidx3d, inv_counts, fc_W,
                      fc_b.reshape(2, 1, _D), reg_W, reg_b.reshape(1, 1))
    return out2d[:, 0]


# speedup vs baseline: 1.4956x; 1.4746x over previous
import functools

import jax
import jax.numpy as jnp
from jax import lax
from jax.experimental import pallas as pl
from jax.experimental.pallas import tpu as pltpu
from jax.experimental.pallas import tpu_sc as plsc

_N_NODES = 10000
_D = 128
_N_GRAPHS = 512
_N_CONV = 3

_BLK = 1000
_NB = _N_NODES // _BLK

_NC = 2
_NS = 16
_CH = 128
_C = 157
_EPAD = _NS * _C * _CH
_HN = 5120
_NPAD = 5376
_RPT = _NPAD // _NS
_DUMMY = _HN


def _softplus(z):
    return jnp.maximum(z, 0.0) + jnp.log1p(jnp.exp(-jnp.abs(z)))


def _sigmoid(z):
    return 1.0 / (1.0 + jnp.exp(-z))


def _dot_t(a, b):
    return lax.dot_general(a, b, (((1,), (1,)), ((), ())),
                           preferred_element_type=jnp.float32)


def _embed_msg_body(nodes_ref, embw_ref, w1_ref, w2_ref, x_ref, m_ref):
    xb = _dot_t(nodes_ref[...], embw_ref[...])
    z1 = _dot_t(xb, w1_ref[...])
    z2 = _dot_t(xb, w2_ref[...])
    x_ref[...] = xb
    m_ref[...] = _sigmoid(z1) * _softplus(z2)


def _update_msg_body(x_ref, agg_ref, w1_ref, w2_ref, xn_ref, m_ref):
    xn = x_ref[...] + _softplus(agg_ref[...])
    z1 = _dot_t(xn, w1_ref[...])
    z2 = _dot_t(xn, w2_ref[...])
    xn_ref[...] = xn
    m_ref[...] = _sigmoid(z1) * _softplus(z2)


def _final_body(x_ref, agg_ref, gidx_ref, invc_ref, fcw_ref, fcb_ref,
                regw_ref, regb_ref, out_ref, acc_ref):
    i = pl.program_id(0)

    @pl.when(i == 0)
    def _():
        acc_ref[...] = jnp.zeros_like(acc_ref)

    xb = x_ref[...] + _softplus(agg_ref[...])
    g = gidx_ref[0]
    iota = lax.broadcasted_iota(jnp.int32, (_N_GRAPHS, _BLK), 0)
    onehot = (iota == g).astype(jnp.float32)
    acc_ref[...] += lax.dot_general(onehot, xb, (((1,), (0,)), ((), ())),
                                    preferred_element_type=jnp.float32)

    @pl.when(i == _NB - 1)
    def _():
        pooled = acc_ref[...] * invc_ref[...]
        y = _softplus(_dot_t(pooled, fcw_ref[0]) + fcb_ref[0])
        y = _softplus(_dot_t(y, fcw_ref[1]) + fcb_ref[1])
        out_ref[...] = (jnp.sum(y * regw_ref[...], axis=1, keepdims=True)
                        + regb_ref[...])


def _tc_embed_msg(nodes, emb_w, w1, w2):
    full = lambda i: (0, 0)
    blk = lambda i: (i, 0)
    return pl.pallas_call(
        _embed_msg_body,
        grid=(_NB,),
        in_specs=[
            pl.BlockSpec((_BLK, _D), blk),
            pl.BlockSpec((_D, _D), full),
            pl.BlockSpec((_D, _D), full),
            pl.BlockSpec((_D, _D), full),
        ],
        out_specs=[
            pl.BlockSpec((_BLK, _D), blk),
            pl.BlockSpec((_BLK, _D), blk),
        ],
        out_shape=[
            jax.ShapeDtypeStruct((_N_NODES, _D), jnp.float32),
            jax.ShapeDtypeStruct((_N_NODES, _D), jnp.float32),
        ],
    )(nodes, emb_w, w1, w2)


def _tc_update_msg(x, agg, w1, w2):
    full = lambda i: (0, 0)
    blk = lambda i: (i, 0)
    return pl.pallas_call(
        _update_msg_body,
        grid=(_NB,),
        in_specs=[
            pl.BlockSpec((_BLK, _D), blk),
            pl.BlockSpec((_BLK, _D), blk),
            pl.BlockSpec((_D, _D), full),
            pl.BlockSpec((_D, _D), full),
        ],
        out_specs=[
            pl.BlockSpec((_BLK, _D), blk),
            pl.BlockSpec((_BLK, _D), blk),
        ],
        out_shape=[
            jax.ShapeDtypeStruct((_N_NODES, _D), jnp.float32),
            jax.ShapeDtypeStruct((_N_NODES, _D), jnp.float32),
        ],
    )(x, agg, w1, w2)


def _tc_final(x, agg, gidx3d, inv_counts, fc_w, fc_b3d, reg_w, reg_b2d):
    blk = lambda i: (i, 0)
    return pl.pallas_call(
        _final_body,
        grid=(_NB,),
        in_specs=[
            pl.BlockSpec((_BLK, _D), blk),
            pl.BlockSpec((_BLK, _D), blk),
            pl.BlockSpec((1, 1, _BLK), lambda i: (i, 0, 0)),
            pl.BlockSpec((_N_GRAPHS, 1), lambda i: (0, 0)),
            pl.BlockSpec((2, _D, _D), lambda i: (0, 0, 0)),
            pl.BlockSpec((2, 1, _D), lambda i: (0, 0, 0)),
            pl.BlockSpec((1, _D), lambda i: (0, 0)),
            pl.BlockSpec((1, 1), lambda i: (0, 0)),
        ],
        out_specs=pl.BlockSpec((_N_GRAPHS, 1), lambda i: (0, 0)),
        out_shape=jax.ShapeDtypeStruct((_N_GRAPHS, 1), jnp.float32),
        scratch_shapes=[pltpu.VMEM((_N_GRAPHS, _D), jnp.float32)],
    )(x, agg, gidx3d, inv_counts, fc_w, fc_b3d, reg_w, reg_b2d)


def _sc_agg(m, zeros_acc, src_idx, tgt_idx):
    mesh = plsc.VectorSubcoreMesh(core_axis_name="c", subcore_axis_name="s")

    @functools.partial(
        pl.kernel,
        out_type=jax.ShapeDtypeStruct((_NC, _NPAD, _D), jnp.float32),
        mesh=mesh,
        scratch_types=[
            pltpu.VMEM((_C, _CH), jnp.int32),
            pltpu.VMEM((_C, _CH), jnp.int32),
            pltpu.VMEM((_CH, _D), jnp.float32),
            pltpu.VMEM((_CH, _D), jnp.float32),
            pltpu.VMEM_SHARED((_NPAD, _D), jnp.float32),
            pltpu.SemaphoreType.DMA,
            pltpu.SemaphoreType.DMA,
        ],
    )
    def k(m_hbm, z_hbm, src_hbm, tgt_hbm, out_hbm, s_v, t_v, rows0_v, rows1_v,
          agg_sh, sem0, sem1):
        cid = lax.axis_index("c")
        sid = lax.axis_index("s")
        base = sid * _RPT

        pltpu.sync_copy(z_hbm.at[pl.ds(base, _RPT)],
                        agg_sh.at[pl.ds(base, _RPT)])

        pltpu.sync_copy(src_hbm.at[sid], s_v)
        pltpu.sync_copy(tgt_hbm.at[cid, sid], t_v)

        plsc.subcore_barrier()

        pltpu.async_copy(m_hbm.at[s_v.at[0]], rows0_v, sem0)

        def step(j, carry):
            for par in range(2):
                rv = rows0_v if par == 0 else rows1_v
                sm = sem0 if par == 0 else sem1
                nrv = rows1_v if par == 0 else rows0_v
                nsm = sem1 if par == 0 else sem0
                jj = 2 * j + par
                pltpu.async_copy(m_hbm.at[s_v.at[jj + 1]], nrv, nsm)
                pltpu.make_async_copy(m_hbm.at[s_v.at[jj]], rv, sm).wait()
                pltpu.sync_copy(rv, agg_sh.at[t_v.at[jj]], add=True)
            return carry
        lax.fori_loop(0, (_C - 1) // 2, step, 0)
        pltpu.make_async_copy(m_hbm.at[s_v.at[_C - 1]], rows0_v, sem0).wait()
        pltpu.sync_copy(rows0_v, agg_sh.at[t_v.at[_C - 1]], add=True)

        plsc.subcore_barrier()

        pltpu.sync_copy(agg_sh.at[pl.ds(base, _RPT)],
                        out_hbm.at[cid].at[pl.ds(base, _RPT)])

    return k(m, zeros_acc, src_idx, tgt_idx)


def kernel(nodes, node_counts, edge_sources, edge_targets, graph_indices,
           emb_W, conv_W1, conv_W2, fc_W, fc_b, reg_W, reg_b):
    pad = _EPAD - edge_sources.shape[0]
    src_p = jnp.concatenate(
        [edge_sources, jnp.zeros((pad,), jnp.int32)]).reshape(_NS, _C, _CH)
    tgt = jnp.concatenate(
        [edge_targets, jnp.full((pad,), _N_NODES, jnp.int32)])
    tgt0 = jnp.where(tgt < _HN, tgt, _DUMMY)
    tgt1 = jnp.where(tgt >= _HN, tgt - _HN, _DUMMY)
    tgt_p = jnp.stack([tgt0, tgt1]).reshape(_NC, _NS, _C, _CH)
    zeros_acc = jnp.zeros((_NPAD, _D), jnp.float32)

    x, m = _tc_embed_msg(nodes, emb_W, conv_W1[0], conv_W2[0])
    for i in range(_N_CONV):
        partials = _sc_agg(m, zeros_acc, src_p, tgt_p)
        agg = jnp.concatenate(
            [partials[0, :_HN], partials[1, :_N_NODES - _HN]])
        if i + 1 < _N_CONV:
            x, m = _tc_update_msg(x, agg, conv_W1[i + 1], conv_W2[i + 1])

    gidx3d = graph_indices.reshape(_NB, 1, _BLK)
    inv_counts = (1.0 / node_counts).reshape(_N_GRAPHS, 1)
    out2d = _tc_final(x, agg, gidx3d, inv_counts, fc_W,
                      fc_b.reshape(2, 1, _D), reg_W, reg_b.reshape(1, 1))
    return out2d[:, 0]


# speedup vs baseline: 2.6080x; 1.7438x over previous
import functools

import jax
import jax.numpy as jnp
from jax import lax
from jax.experimental import pallas as pl
from jax.experimental.pallas import tpu as pltpu
from jax.experimental.pallas import tpu_sc as plsc

_N_NODES = 10000
_D = 128
_N_GRAPHS = 512
_N_CONV = 3

_BLK = 1000
_NB = _N_NODES // _BLK

_NC = 2
_NS = 16
_CH = 128
_C = 157
_EPAD = _NS * _C * _CH
_HN = 5120
_NPAD = 5376
_RPT = _NPAD // _NS
_HN1 = _N_NODES - _HN


def _softplus(z):
    return jnp.maximum(z, 0.0) + jnp.log1p(jnp.exp(-jnp.abs(z)))


def _sigmoid(z):
    return 1.0 / (1.0 + jnp.exp(-z))


def _dot_t(a, b):
    return lax.dot_general(a, b, (((1,), (1,)), ((), ())),
                           preferred_element_type=jnp.float32)


def _embed_msg_body(nodes_ref, embw_ref, w1_ref, w2_ref, x_ref, m_ref):
    xb = _dot_t(nodes_ref[...], embw_ref[...])
    z1 = _dot_t(xb, w1_ref[...])
    z2 = _dot_t(xb, w2_ref[...])
    x_ref[...] = xb
    m_ref[...] = _sigmoid(z1) * _softplus(z2)


def _update_msg_body(x_ref, agg_ref, w1_ref, w2_ref, xn_ref, m_ref):
    xn = x_ref[...] + _softplus(agg_ref[...])
    z1 = _dot_t(xn, w1_ref[...])
    z2 = _dot_t(xn, w2_ref[...])
    xn_ref[...] = xn
    m_ref[...] = _sigmoid(z1) * _softplus(z2)


def _final_body(x_ref, agg_ref, gidx_ref, invc_ref, fcw_ref, fcb_ref,
                regw_ref, regb_ref, out_ref, acc_ref):
    i = pl.program_id(0)

    @pl.when(i == 0)
    def _():
        acc_ref[...] = jnp.zeros_like(acc_ref)

    xb = x_ref[...] + _softplus(agg_ref[...])
    g = gidx_ref[0]
    iota = lax.broadcasted_iota(jnp.int32, (_N_GRAPHS, _BLK), 0)
    onehot = (iota == g).astype(jnp.float32)
    acc_ref[...] += lax.dot_general(onehot, xb, (((1,), (0,)), ((), ())),
                                    preferred_element_type=jnp.float32)

    @pl.when(i == _NB - 1)
    def _():
        pooled = acc_ref[...] * invc_ref[...]
        y = _softplus(_dot_t(pooled, fcw_ref[0]) + fcb_ref[0])
        y = _softplus(_dot_t(y, fcw_ref[1]) + fcb_ref[1])
        out_ref[...] = (jnp.sum(y * regw_ref[...], axis=1, keepdims=True)
                        + regb_ref[...])


def _tc_embed_msg(nodes, emb_w, w1, w2):
    full = lambda i: (0, 0)
    blk = lambda i: (i, 0)
    return pl.pallas_call(
        _embed_msg_body,
        grid=(_NB,),
        in_specs=[
            pl.BlockSpec((_BLK, _D), blk),
            pl.BlockSpec((_D, _D), full),
            pl.BlockSpec((_D, _D), full),
            pl.BlockSpec((_D, _D), full),
        ],
        out_specs=[
            pl.BlockSpec((_BLK, _D), blk),
            pl.BlockSpec((_BLK, _D), blk),
        ],
        out_shape=[
            jax.ShapeDtypeStruct((_N_NODES, _D), jnp.float32),
            jax.ShapeDtypeStruct((_N_NODES, _D), jnp.float32),
        ],
    )(nodes, emb_w, w1, w2)


def _tc_update_msg(x, agg, w1, w2):
    full = lambda i: (0, 0)
    blk = lambda i: (i, 0)
    return pl.pallas_call(
        _update_msg_body,
        grid=(_NB,),
        in_specs=[
            pl.BlockSpec((_BLK, _D), blk),
            pl.BlockSpec((_BLK, _D), blk),
            pl.BlockSpec((_D, _D), full),
            pl.BlockSpec((_D, _D), full),
        ],
        out_specs=[
            pl.BlockSpec((_BLK, _D), blk),
            pl.BlockSpec((_BLK, _D), blk),
        ],
        out_shape=[
            jax.ShapeDtypeStruct((_N_NODES, _D), jnp.float32),
            jax.ShapeDtypeStruct((_N_NODES, _D), jnp.float32),
        ],
    )(x, agg, w1, w2)


def _tc_final(x, agg, gidx3d, inv_counts, fc_w, fc_b3d, reg_w, reg_b2d):
    blk = lambda i: (i, 0)
    return pl.pallas_call(
        _final_body,
        grid=(_NB,),
        in_specs=[
            pl.BlockSpec((_BLK, _D), blk),
            pl.BlockSpec((_BLK, _D), blk),
            pl.BlockSpec((1, 1, _BLK), lambda i: (i, 0, 0)),
            pl.BlockSpec((_N_GRAPHS, 1), lambda i: (0, 0)),
            pl.BlockSpec((2, _D, _D), lambda i: (0, 0, 0)),
            pl.BlockSpec((2, 1, _D), lambda i: (0, 0, 0)),
            pl.BlockSpec((1, _D), lambda i: (0, 0)),
            pl.BlockSpec((1, 1), lambda i: (0, 0)),
        ],
        out_specs=pl.BlockSpec((_N_GRAPHS, 1), lambda i: (0, 0)),
        out_shape=jax.ShapeDtypeStruct((_N_GRAPHS, 1), jnp.float32),
        scratch_shapes=[pltpu.VMEM((_N_GRAPHS, _D), jnp.float32)],
    )(x, agg, gidx3d, inv_counts, fc_w, fc_b3d, reg_w, reg_b2d)


def _sc_agg(m, zeros_acc, src_idx, tgt_idx):
    mesh = plsc.VectorSubcoreMesh(core_axis_name="c", subcore_axis_name="s")

    @functools.partial(
        pl.kernel,
        out_type=jax.ShapeDtypeStruct((_NC, _NPAD, _D), jnp.float32),
        mesh=mesh,
        scratch_types=[
            pltpu.VMEM((_C, _CH), jnp.int32),
            pltpu.VMEM((_C, _CH), jnp.int32),
            pltpu.VMEM((_CH, _D), jnp.float32),
            pltpu.VMEM((_CH, _D), jnp.float32),
            pltpu.VMEM_SHARED((_NPAD, _D), jnp.float32),
            pltpu.SemaphoreType.DMA,
            pltpu.SemaphoreType.DMA,
        ],
    )
    def k(m_hbm, z_hbm, src_hbm, tgt_hbm, out_hbm, s_v, t_v, rows0_v, rows1_v,
          agg_sh, sem0, sem1):
        cid = lax.axis_index("c")
        sid = lax.axis_index("s")
        base = sid * _RPT

        pltpu.sync_copy(z_hbm.at[pl.ds(base, _RPT)],
                        agg_sh.at[pl.ds(base, _RPT)])

        pltpu.sync_copy(src_hbm.at[sid], s_v)
        pltpu.sync_copy(tgt_hbm.at[cid, sid], t_v)

        plsc.subcore_barrier()

        pltpu.async_copy(m_hbm.at[s_v.at[0]], rows0_v, sem0)

        def step(j, carry):
            for par in range(2):
                rv = rows0_v if par == 0 else rows1_v
                sm = sem0 if par == 0 else sem1
                nrv = rows1_v if par == 0 else rows0_v
                nsm = sem1 if par == 0 else sem0
                jj = 2 * j + par
                pltpu.async_copy(m_hbm.at[s_v.at[jj + 1]], nrv, nsm)
                pltpu.make_async_copy(m_hbm.at[s_v.at[jj]], rv, sm).wait()
                pltpu.sync_copy(rv, agg_sh.at[t_v.at[jj]], add=True)
            return carry
        lax.fori_loop(0, (_C - 1) // 2, step, 0)
        pltpu.make_async_copy(m_hbm.at[s_v.at[_C - 1]], rows0_v, sem0).wait()
        pltpu.sync_copy(rows0_v, agg_sh.at[t_v.at[_C - 1]], add=True)

        plsc.subcore_barrier()

        pltpu.sync_copy(agg_sh.at[pl.ds(base, _RPT)],
                        out_hbm.at[cid].at[pl.ds(base, _RPT)])

    return k(m, zeros_acc, src_idx, tgt_idx)


def kernel(nodes, node_counts, edge_sources, edge_targets, graph_indices,
           emb_W, conv_W1, conv_W2, fc_W, fc_b, reg_W, reg_b):
    pad = _EPAD - edge_sources.shape[0]
    eidx = jnp.arange(_EPAD, dtype=jnp.int32)
    src_p = jnp.concatenate(
        [edge_sources,
         eidx[:pad] % _N_NODES]).reshape(_NS, _C, _CH)
    tgt = jnp.concatenate(
        [edge_targets, jnp.full((pad,), _N_NODES, jnp.int32)])
    tgt0 = jnp.where(tgt < _HN, tgt, _HN + (eidx % (_NPAD - _HN)))
    tgt1 = jnp.where((tgt >= _HN) & (tgt < _N_NODES), tgt - _HN,
                     _HN1 + (eidx % (_NPAD - _HN1)))
    tgt_p = jnp.stack([tgt0, tgt1]).reshape(_NC, _NS, _C, _CH)
    zeros_acc = jnp.zeros((_NPAD, _D), jnp.float32)

    x, m = _tc_embed_msg(nodes, emb_W, conv_W1[0], conv_W2[0])
    for i in range(_N_CONV):
        partials = _sc_agg(m, zeros_acc, src_p, tgt_p)
        agg = jnp.concatenate(
            [partials[0, :_HN], partials[1, :_N_NODES - _HN]])
        if i + 1 < _N_CONV:
            x, m = _tc_update_msg(x, agg, conv_W1[i + 1], conv_W2[i + 1])

    gidx3d = graph_indices.reshape(_NB, 1, _BLK)
    inv_counts = (1.0 / node_counts).reshape(_N_GRAPHS, 1)
    out2d = _tc_final(x, agg, gidx3d, inv_counts, fc_W,
                      fc_b.reshape(2, 1, _D), reg_W, reg_b.reshape(1, 1))
    return out2d[:, 0]
